# baseline (speedup 1.0000x reference)
import functools

import jax
import jax.numpy as jnp
from jax import lax
from jax.experimental import pallas as pl
from jax.experimental.pallas import tpu as pltpu
from jax.experimental.pallas import tpu_sc as plsc

_B = 16384
_F = 26
_E = 32
_BUCKET = 100000
_NC = 2
_NS = 16
_NW = _NC * _NS
_L = 16
_H0 = 50048
_H1M = 49920
_TAIL = _BUCKET - _H0 - _H1M
_Q = 4096
_NQ = _B // _Q


@functools.cache
def _sc_gather_fn():
    mesh = plsc.VectorSubcoreMesh(
        core_axis_name="c", subcore_axis_name="s", num_cores=_NC,
        num_subcores=_NS)

    @functools.partial(
        pl.kernel,
        out_type=(
            jax.ShapeDtypeStruct((_F * _E, _B), jnp.float32),
            jax.ShapeDtypeStruct((_F, _B), jnp.float32),
        ),
        mesh=mesh,
        scratch_types=[
            pltpu.VMEM((_H0,), jnp.float32),
            pltpu.VMEM((_H0,), jnp.float32),
            pltpu.VMEM((2, _Q), jnp.int32),
            pltpu.VMEM((_B,), jnp.float32),
            pltpu.SemaphoreType.DMA,
            pltpu.SemaphoreType.DMA,
            pltpu.SemaphoreType.DMA,
            pltpu.SemaphoreType.DMA,
        ],
        compiler_params=pltpu.CompilerParams(
            use_tc_tiling_on_sc=True, needs_layout_passes=False),
    )
    def sc_gather(feats_hbm, tt_hbm, ttail_hbm, wt_hbm, wtail_hbm,
                  emb_out, wide_out,
                  buf_a, buf_b, idxq, part, sem_a, sem_b, sem_i, sem_o):
        e = lax.axis_index("c") * _NS + lax.axis_index("s")

        def row_a(f):
            return pltpu.make_async_copy(
                tt_hbm.at[f * _E + e, pl.ds(0, _H0)], buf_a, sem_a)

        def row_b_main(f):
            return pltpu.make_async_copy(
                tt_hbm.at[f * _E + e, pl.ds(_H0, _H1M)],
                buf_b.at[pl.ds(0, _H1M)], sem_b)

        def row_b_tail(f):
            return pltpu.make_async_copy(
                ttail_hbm.at[f * _E + e],
                buf_b.at[pl.ds(_H1M, 128)], sem_b)

        def idx_copy(f, q, s):
            return pltpu.make_async_copy(
                feats_hbm.at[f, pl.ds(q * _Q, _Q)], idxq.at[s], sem_i)

        def out_copy(row, q):
            return pltpu.make_async_copy(
                part.at[pl.ds(q * _Q, _Q)],
                emb_out.at[row, pl.ds(q * _Q, _Q)], sem_o)

        def pass1_quarter(s, qbase):
            @plsc.parallel_loop(0, _Q // _L, unroll=4)
            def _loop(k):
                ids = idxq[s, pl.ds(k * _L, _L)]
                part[pl.ds(qbase + k * _L, _L)] = plsc.load_gather(
                    buf_a, [jnp.minimum(ids, _H0 - 1)])

        def pass2_quarter(s, qbase):
            @plsc.parallel_loop(0, _Q // _L, unroll=4)
            def _loop(k):
                ids = idxq[s, pl.ds(k * _L, _L)]
                g2 = plsc.load_gather(
                    buf_b, [jnp.maximum(ids - _H0, 0)])
                old = part[pl.ds(qbase + k * _L, _L)]
                part[pl.ds(qbase + k * _L, _L)] = jnp.where(
                    ids >= _H0, g2, old)

        def field_body(f, carry):
            row = f * _E + e
            row_a(f).wait()

            @pl.when(f > 0)
            def _wait_prev_out():
                for q in range(_NQ):
                    out_copy(row, q).wait()

            for q in range(_NQ):
                idx_copy(f, q, q % 2).wait()
                if q < _NQ - 1:
                    idx_copy(f, q + 1, (q + 1) % 2).start()
                pass1_quarter(q % 2, q * _Q)

            @pl.when(f < _F - 1)
            def _prefetch_a():
                row_a(f + 1).start()

            row_b_main(f).wait()
            row_b_tail(f).wait()
            idx_copy(f, 0, 0).start()
            for q in range(_NQ):
                idx_copy(f, q, q % 2).wait()
                if q < _NQ - 1:
                    idx_copy(f, q + 1, (q + 1) % 2).start()
                pass2_quarter(q % 2, q * _Q)
                out_copy(row, q).start()

            @pl.when(f < _F - 1)
            def _prefetch_b():
                row_b_main(f + 1).start()
                row_b_tail(f + 1).start()
                idx_copy(f + 1, 0, 0).start()

            return carry

        row_a(0).start()
        row_b_main(0).start()
        row_b_tail(0).start()
        idx_copy(0, 0, 0).start()
        lax.fori_loop(0, _F, field_body, 0)
        for q in range(_NQ):
            out_copy((_F - 1) * _E + e, q).wait()

        @pl.when(e < _F)
        def _wide():
            pltpu.sync_copy(wt_hbm.at[e, pl.ds(0, _H0)], buf_a)
            pltpu.sync_copy(wt_hbm.at[e, pl.ds(_H0, _H1M)],
                            buf_b.at[pl.ds(0, _H1M)])
            pltpu.sync_copy(wtail_hbm.at[e], buf_b.at[pl.ds(_H1M, 128)])
            for q in range(_NQ):
                pltpu.sync_copy(feats_hbm.at[e, pl.ds(q * _Q, _Q)],
                                idxq.at[0])

                @plsc.parallel_loop(0, _Q // _L, unroll=4)
                def _loop(k):
                    ids = idxq[0, pl.ds(k * _L, _L)]
                    g1 = plsc.load_gather(
                        buf_a, [jnp.minimum(ids, _H0 - 1)])
                    g2 = plsc.load_gather(
                        buf_b, [jnp.maximum(ids - _H0, 0)])
                    part[pl.ds(q * _Q + k * _L, _L)] = jnp.where(
                        ids >= _H0, g2, g1)
                pltpu.sync_copy(part.at[pl.ds(q * _Q, _Q)],
                                wide_out.at[e, pl.ds(q * _Q, _Q)])

    return sc_gather


def kernel(feat_0, feat_1, feat_2, feat_3, feat_4, feat_5, feat_6, feat_7,
           feat_8, feat_9, feat_10, feat_11, feat_12, feat_13, feat_14,
           feat_15, feat_16, feat_17, feat_18, feat_19, feat_20, feat_21,
           feat_22, feat_23, feat_24, feat_25, emb_tables, wide_weights):
    feats = jnp.stack(
        [feat_0[:, 0], feat_1[:, 0], feat_2[:, 0], feat_3[:, 0],
         feat_4[:, 0], feat_5[:, 0], feat_6[:, 0], feat_7[:, 0],
         feat_8[:, 0], feat_9[:, 0], feat_10[:, 0], feat_11[:, 0],
         feat_12[:, 0], feat_13[:, 0], feat_14[:, 0], feat_15[:, 0],
         feat_16[:, 0], feat_17[:, 0], feat_18[:, 0], feat_19[:, 0],
         feat_20[:, 0], feat_21[:, 0], feat_22[:, 0], feat_23[:, 0],
         feat_24[:, 0], feat_25[:, 0]], axis=0)
    tt = emb_tables.transpose(0, 2, 1).reshape(_F * _E, _BUCKET)
    ttail = jnp.pad(tt[:, _H0 + _H1M:], ((0, 0), (0, 128 - _TAIL)))
    wtail = jnp.pad(wide_weights[:, _H0 + _H1M:], ((0, 0), (0, 128 - _TAIL)))
    emb_t, wide_t = _sc_gather_fn()(feats, tt, ttail, wide_weights, wtail)
    emb = emb_t.reshape(_F, _E, _B).transpose(2, 0, 1)
    wide = wide_t.transpose(1, 0)
    return (wide, emb)


# speedup vs baseline: 1.0321x; 1.0321x over previous
import functools

import jax
import jax.numpy as jnp
from jax import lax
from jax.experimental import pallas as pl
from jax.experimental.pallas import tpu as pltpu
from jax.experimental.pallas import tpu_sc as plsc

_B = 16384
_F = 26
_E = 32
_BUCKET = 100000
_NC = 2
_NS = 16
_NW = _NC * _NS
_L = 16
_H0 = 50048
_H1M = 49920
_TAIL = _BUCKET - _H0 - _H1M
_Q = 4096
_NQ = _B // _Q


@functools.cache
def _sc_gather_fn():
    mesh = plsc.VectorSubcoreMesh(
        core_axis_name="c", subcore_axis_name="s", num_cores=_NC,
        num_subcores=_NS)

    @functools.partial(
        pl.kernel,
        out_type=(
            jax.ShapeDtypeStruct((_F * _E, _B), jnp.float32),
            jax.ShapeDtypeStruct((_F, _B), jnp.float32),
        ),
        mesh=mesh,
        scratch_types=[
            pltpu.VMEM((_H0,), jnp.float32),
            pltpu.VMEM((_H0,), jnp.float32),
            pltpu.VMEM((2, _Q), jnp.int32),
            pltpu.VMEM((_B,), jnp.float32),
            pltpu.SemaphoreType.DMA,
            pltpu.SemaphoreType.DMA,
            pltpu.SemaphoreType.DMA,
            pltpu.SemaphoreType.DMA,
        ],
        compiler_params=pltpu.CompilerParams(
            use_tc_tiling_on_sc=True, needs_layout_passes=False),
    )
    def sc_gather(feats_hbm, tt_hbm, ttail_hbm, wt_hbm, wtail_hbm,
                  emb_out, wide_out,
                  buf_a, buf_b, idxq, part, sem_a, sem_b, sem_i, sem_o):
        e = lax.axis_index("c") * _NS + lax.axis_index("s")

        def row_a(f):
            return pltpu.make_async_copy(
                tt_hbm.at[f * _E + e, pl.ds(0, _H0)], buf_a, sem_a)

        def row_b_main(f):
            return pltpu.make_async_copy(
                tt_hbm.at[f * _E + e, pl.ds(_H0, _H1M)],
                buf_b.at[pl.ds(0, _H1M)], sem_b)

        def row_b_tail(f):
            return pltpu.make_async_copy(
                ttail_hbm.at[f * _E + e],
                buf_b.at[pl.ds(_H1M, 128)], sem_b)

        def idx_copy(f, q, s):
            return pltpu.make_async_copy(
                feats_hbm.at[f, pl.ds(q * _Q, _Q)], idxq.at[s], sem_i)

        def out_copy(row, q):
            return pltpu.make_async_copy(
                part.at[pl.ds(q * _Q, _Q)],
                emb_out.at[row, pl.ds(q * _Q, _Q)], sem_o)

        lane = lax.iota(jnp.int32, _L)

        def pass1_quarter(s, qbase):
            @plsc.parallel_loop(0, _Q // _L, unroll=4)
            def _loop(k):
                ids = idxq[s, pl.ds(k * _L, _L)]
                m = ids < _H0
                g1 = plsc.load_gather(buf_a, [ids], mask=m)
                plsc.store_scatter(part, [qbase + k * _L + lane], g1, mask=m)

        def pass2_quarter(s, qbase):
            @plsc.parallel_loop(0, _Q // _L, unroll=4)
            def _loop(k):
                ids = idxq[s, pl.ds(k * _L, _L)]
                m = ids >= _H0
                g2 = plsc.load_gather(buf_b, [ids - _H0], mask=m)
                plsc.store_scatter(part, [qbase + k * _L + lane], g2, mask=m)

        def field_body(f, carry):
            row = f * _E + e
            row_a(f).wait()

            @pl.when(f > 0)
            def _wait_prev_out():
                for q in range(_NQ):
                    out_copy(row, q).wait()

            for q in range(_NQ):
                idx_copy(f, q, q % 2).wait()
                if q < _NQ - 1:
                    idx_copy(f, q + 1, (q + 1) % 2).start()
                pass1_quarter(q % 2, q * _Q)

            @pl.when(f < _F - 1)
            def _prefetch_a():
                row_a(f + 1).start()

            row_b_main(f).wait()
            row_b_tail(f).wait()
            idx_copy(f, 0, 0).start()
            for q in range(_NQ):
                idx_copy(f, q, q % 2).wait()
                if q < _NQ - 1:
                    idx_copy(f, q + 1, (q + 1) % 2).start()
                pass2_quarter(q % 2, q * _Q)
                out_copy(row, q).start()

            @pl.when(f < _F - 1)
            def _prefetch_b():
                row_b_main(f + 1).start()
                row_b_tail(f + 1).start()
                idx_copy(f + 1, 0, 0).start()

            return carry

        row_a(0).start()
        row_b_main(0).start()
        row_b_tail(0).start()
        idx_copy(0, 0, 0).start()
        lax.fori_loop(0, _F, field_body, 0)
        for q in range(_NQ):
            out_copy((_F - 1) * _E + e, q).wait()

        @pl.when(e < _F)
        def _wide():
            pltpu.sync_copy(wt_hbm.at[e, pl.ds(0, _H0)], buf_a)
            pltpu.sync_copy(wt_hbm.at[e, pl.ds(_H0, _H1M)],
                            buf_b.at[pl.ds(0, _H1M)])
            pltpu.sync_copy(wtail_hbm.at[e], buf_b.at[pl.ds(_H1M, 128)])
            for q in range(_NQ):
                pltpu.sync_copy(feats_hbm.at[e, pl.ds(q * _Q, _Q)],
                                idxq.at[0])

                @plsc.parallel_loop(0, _Q // _L, unroll=4)
                def _loop(k):
                    ids = idxq[0, pl.ds(k * _L, _L)]
                    g1 = plsc.load_gather(
                        buf_a, [jnp.minimum(ids, _H0 - 1)])
                    g2 = plsc.load_gather(
                        buf_b, [jnp.maximum(ids - _H0, 0)])
                    part[pl.ds(q * _Q + k * _L, _L)] = jnp.where(
                        ids >= _H0, g2, g1)
                pltpu.sync_copy(part.at[pl.ds(q * _Q, _Q)],
                                wide_out.at[e, pl.ds(q * _Q, _Q)])

    return sc_gather


def kernel(feat_0, feat_1, feat_2, feat_3, feat_4, feat_5, feat_6, feat_7,
           feat_8, feat_9, feat_10, feat_11, feat_12, feat_13, feat_14,
           feat_15, feat_16, feat_17, feat_18, feat_19, feat_20, feat_21,
           feat_22, feat_23, feat_24, feat_25, emb_tables, wide_weights):
    feats = jnp.stack(
        [feat_0[:, 0], feat_1[:, 0], feat_2[:, 0], feat_3[:, 0],
         feat_4[:, 0], feat_5[:, 0], feat_6[:, 0], feat_7[:, 0],
         feat_8[:, 0], feat_9[:, 0], feat_10[:, 0], feat_11[:, 0],
         feat_12[:, 0], feat_13[:, 0], feat_14[:, 0], feat_15[:, 0],
         feat_16[:, 0], feat_17[:, 0], feat_18[:, 0], feat_19[:, 0],
         feat_20[:, 0], feat_21[:, 0], feat_22[:, 0], feat_23[:, 0],
         feat_24[:, 0], feat_25[:, 0]], axis=0)
    tt = emb_tables.transpose(0, 2, 1).reshape(_F * _E, _BUCKET)
    ttail = jnp.pad(tt[:, _H0 + _H1M:], ((0, 0), (0, 128 - _TAIL)))
    wtail = jnp.pad(wide_weights[:, _H0 + _H1M:], ((0, 0), (0, 128 - _TAIL)))
    emb_t, wide_t = _sc_gather_fn()(feats, tt, ttail, wide_weights, wtail)
    emb = emb_t.reshape(_F, _E, _B).transpose(2, 0, 1)
    wide = wide_t.transpose(1, 0)
    return (wide, emb)


# speedup vs baseline: 1.3117x; 1.2709x over previous
import functools

import jax
import jax.numpy as jnp
from jax import lax
from jax.experimental import pallas as pl
from jax.experimental.pallas import tpu as pltpu
from jax.experimental.pallas import tpu_sc as plsc

_B = 16384
_F = 26
_E = 32
_BUCKET = 100000
_NC = 2
_NS = 16
_NW = _NC * _NS
_L = 16
_Q = 4096
_NQ = _B // _Q
_CH = 25088
_MAIN = 3 * _CH + 24704
_PAD = _BUCKET - _MAIN


@functools.cache
def _sc_gather_fn():
    mesh = plsc.VectorSubcoreMesh(
        core_axis_name="c", subcore_axis_name="s", num_cores=_NC,
        num_subcores=_NS)

    @functools.partial(
        pl.kernel,
        out_type=(
            jax.ShapeDtypeStruct((_F * _E, _B), jnp.float32),
            jax.ShapeDtypeStruct((_F, _B), jnp.float32),
        ),
        mesh=mesh,
        scratch_types=[
            pltpu.VMEM((_MAIN + 128,), jnp.float32),
            pltpu.VMEM((2, _Q), jnp.int32),
            pltpu.VMEM((_B,), jnp.float32),
            pltpu.SemaphoreType.DMA,
            pltpu.SemaphoreType.DMA,
            pltpu.SemaphoreType.DMA,
        ],
        compiler_params=pltpu.CompilerParams(
            use_tc_tiling_on_sc=True, needs_layout_passes=False),
    )
    def sc_gather(feats_hbm, tt_hbm, ttail_hbm, wt_hbm, wtail_hbm,
                  emb_out, wide_out, rowbuf, idxq, part, sem_r, sem_i, sem_o):
        e = lax.axis_index("c") * _NS + lax.axis_index("s")

        def row_copies(src_hbm, tail_hbm, r):
            cps = []
            for off in range(0, _MAIN, _CH):
                ln = min(_CH, _MAIN - off)
                cps.append(pltpu.make_async_copy(
                    src_hbm.at[r, pl.ds(off, ln)],
                    rowbuf.at[pl.ds(off, ln)], sem_r))
            cps.append(pltpu.make_async_copy(
                tail_hbm.at[r], rowbuf.at[pl.ds(_MAIN, 128)], sem_r))
            return cps

        def idx_copy(f, q, s):
            return pltpu.make_async_copy(
                feats_hbm.at[f, pl.ds(q * _Q, _Q)], idxq.at[s], sem_i)

        def out_copy(row, q):
            return pltpu.make_async_copy(
                part.at[pl.ds(q * _Q, _Q)],
                emb_out.at[row, pl.ds(q * _Q, _Q)], sem_o)

        def gather_quarter(s, qbase):
            @plsc.parallel_loop(0, _Q // _L, unroll=8)
            def _loop(k):
                ids = idxq[s, pl.ds(k * _L, _L)]
                part[pl.ds(qbase + k * _L, _L)] = plsc.load_gather(
                    rowbuf, [ids])

        def field_body(f, carry):
            row = f * _E + e
            for cp in row_copies(tt_hbm, ttail_hbm, row):
                cp.start()

            @pl.when(f > 0)
            def _wait_prev_out():
                for q in range(_NQ):
                    out_copy(row, q).wait()

            for cp in row_copies(tt_hbm, ttail_hbm, row):
                cp.wait()
            for q in range(_NQ):
                idx_copy(f, q, q % 2).wait()
                if q < _NQ - 1:
                    idx_copy(f, q + 1, (q + 1) % 2).start()
                gather_quarter(q % 2, q * _Q)
                out_copy(row, q).start()

            @pl.when(f < _F - 1)
            def _prefetch_idx():
                idx_copy(f + 1, 0, 0).start()

            return carry

        idx_copy(0, 0, 0).start()
        lax.fori_loop(0, _F, field_body, 0)
        for q in range(_NQ):
            out_copy((_F - 1) * _E + e, q).wait()

        @pl.when(e < _F)
        def _wide():
            for cp in row_copies(wt_hbm, wtail_hbm, e):
                cp.start()
            for cp in row_copies(wt_hbm, wtail_hbm, e):
                cp.wait()
            for q in range(_NQ):
                pltpu.sync_copy(feats_hbm.at[e, pl.ds(q * _Q, _Q)],
                                idxq.at[0])
                gather_quarter(0, q * _Q)
                pltpu.sync_copy(part.at[pl.ds(q * _Q, _Q)],
                                wide_out.at[e, pl.ds(q * _Q, _Q)])

    return sc_gather


def kernel(feat_0, feat_1, feat_2, feat_3, feat_4, feat_5, feat_6, feat_7,
           feat_8, feat_9, feat_10, feat_11, feat_12, feat_13, feat_14,
           feat_15, feat_16, feat_17, feat_18, feat_19, feat_20, feat_21,
           feat_22, feat_23, feat_24, feat_25, emb_tables, wide_weights):
    feats = jnp.stack(
        [feat_0[:, 0], feat_1[:, 0], feat_2[:, 0], feat_3[:, 0],
         feat_4[:, 0], feat_5[:, 0], feat_6[:, 0], feat_7[:, 0],
         feat_8[:, 0], feat_9[:, 0], feat_10[:, 0], feat_11[:, 0],
         feat_12[:, 0], feat_13[:, 0], feat_14[:, 0], feat_15[:, 0],
         feat_16[:, 0], feat_17[:, 0], feat_18[:, 0], feat_19[:, 0],
         feat_20[:, 0], feat_21[:, 0], feat_22[:, 0], feat_23[:, 0],
         feat_24[:, 0], feat_25[:, 0]], axis=0)
    tt = emb_tables.transpose(0, 2, 1).reshape(_F * _E, _BUCKET)
    ttail = jnp.pad(tt[:, _MAIN:], ((0, 0), (0, 128 - _PAD)))
    wtail = jnp.pad(wide_weights[:, _MAIN:], ((0, 0), (0, 128 - _PAD)))
    emb_t, wide_t = _sc_gather_fn()(feats, tt, ttail, wide_weights, wtail)
    emb = emb_t.reshape(_F, _E, _B).transpose(2, 0, 1)
    wide = wide_t.transpose(1, 0)
    return (wide, emb)
